# baseline (speedup 1.0000x reference)
import functools

import jax
import jax.numpy as jnp
from jax.experimental import pallas as pl
from jax.experimental.pallas import tpu as pltpu


def _moe_dense_body(x_ref, gk_ref, w0_ref, w1_ref, wo_ref, out_ref,
                    acc_ref, cmb_ref, *, n_e, n_f):
    e = pl.program_id(1)
    f = pl.program_id(2)

    @pl.when(jnp.logical_and(e == 0, f == 0))
    def _gate():
        x = x_ref[...]
        logits = jax.lax.dot_general(
            x, gk_ref[...], (((1,), (0,)), ((), ())),
            preferred_element_type=jnp.float32)
        a1 = jnp.argmax(logits, axis=-1)
        m1 = jnp.max(logits, axis=-1)
        ids = jax.lax.broadcasted_iota(jnp.int32, logits.shape, 1)
        logits2 = jnp.where(ids == a1[:, None], -jnp.inf, logits)
        a2 = jnp.argmax(logits2, axis=-1)
        m2 = jnp.max(logits2, axis=-1)
        p1 = 1.0 / (1.0 + jnp.exp(m2 - m1))
        p2 = 1.0 - p1
        cmb = (jnp.where(ids == a1[:, None], p1[:, None], 0.0)
               + jnp.where(ids == a2[:, None], p2[:, None], 0.0))
        cmb_ref[...] = cmb
        acc_ref[...] = jnp.zeros_like(acc_ref)

    x = x_ref[...]
    h0 = jax.lax.dot_general(x, w0_ref[0], (((1,), (0,)), ((), ())),
                             preferred_element_type=jnp.float32)
    h1 = jax.lax.dot_general(x, w1_ref[0], (((1,), (0,)), ((), ())),
                             preferred_element_type=jnp.float32)
    g = (h0 * jax.nn.sigmoid(h0)) * h1
    part = jax.lax.dot_general(g, wo_ref[0], (((1,), (0,)), ((), ())),
                               preferred_element_type=jnp.float32)
    cmb = cmb_ref[...]
    ids = jax.lax.broadcasted_iota(jnp.int32, cmb.shape, 1)
    w_e = jnp.sum(jnp.where(ids == e, cmb, 0.0), axis=1, keepdims=True)
    acc_ref[...] += w_e * part

    @pl.when(jnp.logical_and(e == n_e - 1, f == n_f - 1))
    def _flush():
        out_ref[...] = acc_ref[...]


def kernel(x, gate_kernel, w0, w1, wo):
    b, s, d = x.shape
    n_e, _, d_ff = w0.shape
    xs = x.reshape(b * s, d)

    tm = 256
    tf = 512
    n_tm = (b * s) // tm
    n_f = d_ff // tf

    out = pl.pallas_call(
        functools.partial(_moe_dense_body, n_e=n_e, n_f=n_f),
        grid=(n_tm, n_e, n_f),
        in_specs=[
            pl.BlockSpec((tm, d), lambda i, e, f: (i, 0)),
            pl.BlockSpec((d, n_e), lambda i, e, f: (0, 0)),
            pl.BlockSpec((1, d, tf), lambda i, e, f: (e, 0, f)),
            pl.BlockSpec((1, d, tf), lambda i, e, f: (e, 0, f)),
            pl.BlockSpec((1, tf, d), lambda i, e, f: (e, f, 0)),
        ],
        out_specs=pl.BlockSpec((tm, d), lambda i, e, f: (i, 0)),
        out_shape=jax.ShapeDtypeStruct((b * s, d), jnp.float32),
        scratch_shapes=[
            pltpu.VMEM((tm, d), jnp.float32),
            pltpu.VMEM((tm, n_e), jnp.float32),
        ],
        compiler_params=pltpu.CompilerParams(
            dimension_semantics=("parallel", "arbitrary", "arbitrary"),
        ),
    )(xs, gate_kernel, w0, w1, wo)
    return out.reshape(b, s, d)


# speedup vs baseline: 1.0818x; 1.0818x over previous
import functools

import jax
import jax.numpy as jnp
from jax.experimental import pallas as pl
from jax.experimental.pallas import tpu as pltpu


def _moe_dense_body(x_ref, gk_ref, w0_ref, w1_ref, wo_ref, out_ref,
                    acc_ref, cmb_ref, *, n_e, n_f):
    e = pl.program_id(1)
    f = pl.program_id(2)

    @pl.when(jnp.logical_and(e == 0, f == 0))
    def _gate():
        x = x_ref[...]
        logits = jax.lax.dot_general(
            x, gk_ref[...], (((1,), (0,)), ((), ())),
            preferred_element_type=jnp.float32)
        a1 = jnp.argmax(logits, axis=-1)
        m1 = jnp.max(logits, axis=-1)
        ids = jax.lax.broadcasted_iota(jnp.int32, logits.shape, 1)
        logits2 = jnp.where(ids == a1[:, None], -jnp.inf, logits)
        a2 = jnp.argmax(logits2, axis=-1)
        m2 = jnp.max(logits2, axis=-1)
        p1 = 1.0 / (1.0 + jnp.exp(m2 - m1))
        p2 = 1.0 - p1
        cmb = (jnp.where(ids == a1[:, None], p1[:, None], 0.0)
               + jnp.where(ids == a2[:, None], p2[:, None], 0.0))
        cmb_ref[...] = cmb
        acc_ref[...] = jnp.zeros_like(acc_ref)

    x = x_ref[...].astype(jnp.bfloat16)
    h0 = jax.lax.dot_general(x, w0_ref[0], (((1,), (0,)), ((), ())),
                             preferred_element_type=jnp.float32)
    h1 = jax.lax.dot_general(x, w1_ref[0], (((1,), (0,)), ((), ())),
                             preferred_element_type=jnp.float32)
    g = ((h0 * jax.nn.sigmoid(h0)) * h1).astype(jnp.bfloat16)
    part = jax.lax.dot_general(g, wo_ref[0], (((1,), (0,)), ((), ())),
                               preferred_element_type=jnp.float32)
    cmb = cmb_ref[...]
    ids = jax.lax.broadcasted_iota(jnp.int32, cmb.shape, 1)
    w_e = jnp.sum(jnp.where(ids == e, cmb, 0.0), axis=1, keepdims=True)
    acc_ref[...] += w_e * part

    @pl.when(jnp.logical_and(e == n_e - 1, f == n_f - 1))
    def _flush():
        out_ref[...] = acc_ref[...]


def kernel(x, gate_kernel, w0, w1, wo):
    b, s, d = x.shape
    n_e, _, d_ff = w0.shape
    xs = x.reshape(b * s, d)
    w0 = w0.astype(jnp.bfloat16)
    w1 = w1.astype(jnp.bfloat16)
    wo = wo.astype(jnp.bfloat16)

    tm = 256
    tf = 512
    n_tm = (b * s) // tm
    n_f = d_ff // tf

    out = pl.pallas_call(
        functools.partial(_moe_dense_body, n_e=n_e, n_f=n_f),
        grid=(n_tm, n_e, n_f),
        in_specs=[
            pl.BlockSpec((tm, d), lambda i, e, f: (i, 0)),
            pl.BlockSpec((d, n_e), lambda i, e, f: (0, 0)),
            pl.BlockSpec((1, d, tf), lambda i, e, f: (e, 0, f)),
            pl.BlockSpec((1, d, tf), lambda i, e, f: (e, 0, f)),
            pl.BlockSpec((1, tf, d), lambda i, e, f: (e, f, 0)),
        ],
        out_specs=pl.BlockSpec((tm, d), lambda i, e, f: (i, 0)),
        out_shape=jax.ShapeDtypeStruct((b * s, d), jnp.float32),
        scratch_shapes=[
            pltpu.VMEM((tm, d), jnp.float32),
            pltpu.VMEM((tm, n_e), jnp.float32),
        ],
        compiler_params=pltpu.CompilerParams(
            dimension_semantics=("parallel", "arbitrary", "arbitrary"),
        ),
    )(xs, gate_kernel, w0, w1, wo)
    return out.reshape(b, s, d)


# speedup vs baseline: 1.7841x; 1.6493x over previous
import functools

import jax
import jax.numpy as jnp
from jax.experimental import pallas as pl
from jax.experimental.pallas import tpu as pltpu


def _moe_dense_body(x_ref, gk_ref, w0_ref, w1_ref, wo_ref, out_ref,
                    acc_ref, cmb_ref, *, n_e, n_f):
    e = pl.program_id(1)
    f = pl.program_id(2)

    @pl.when(jnp.logical_and(e == 0, f == 0))
    def _gate():
        x = x_ref[...]
        logits = jax.lax.dot_general(
            x, gk_ref[...], (((1,), (0,)), ((), ())),
            preferred_element_type=jnp.float32)
        a1 = jnp.argmax(logits, axis=-1)
        m1 = jnp.max(logits, axis=-1)
        ids = jax.lax.broadcasted_iota(jnp.int32, logits.shape, 1)
        logits2 = jnp.where(ids == a1[:, None], -jnp.inf, logits)
        a2 = jnp.argmax(logits2, axis=-1)
        m2 = jnp.max(logits2, axis=-1)
        p1 = 1.0 / (1.0 + jnp.exp(m2 - m1))
        p2 = 1.0 - p1
        cmb = (jnp.where(ids == a1[:, None], p1[:, None], 0.0)
               + jnp.where(ids == a2[:, None], p2[:, None], 0.0))
        cmb_ref[...] = cmb
        acc_ref[...] = jnp.zeros_like(acc_ref)

    x = x_ref[...].astype(jnp.bfloat16)
    h0 = jax.lax.dot_general(x, w0_ref[0], (((1,), (0,)), ((), ())),
                             preferred_element_type=jnp.float32)
    h1 = jax.lax.dot_general(x, w1_ref[0], (((1,), (0,)), ((), ())),
                             preferred_element_type=jnp.float32)
    g = ((h0 * jax.nn.sigmoid(h0)) * h1).astype(jnp.bfloat16)
    part = jax.lax.dot_general(g, wo_ref[0], (((1,), (0,)), ((), ())),
                               preferred_element_type=jnp.float32)
    cmb = cmb_ref[...]
    ids = jax.lax.broadcasted_iota(jnp.int32, cmb.shape, 1)
    w_e = jnp.sum(jnp.where(ids == e, cmb, 0.0), axis=1, keepdims=True)
    acc_ref[...] += w_e * part

    @pl.when(jnp.logical_and(e == n_e - 1, f == n_f - 1))
    def _flush():
        out_ref[...] = acc_ref[...]


def kernel(x, gate_kernel, w0, w1, wo):
    b, s, d = x.shape
    n_e, _, d_ff = w0.shape
    xs = x.reshape(b * s, d)
    w0 = w0.astype(jnp.bfloat16)
    w1 = w1.astype(jnp.bfloat16)
    wo = wo.astype(jnp.bfloat16)

    tm = 2048
    tf = 512
    n_tm = (b * s) // tm
    n_f = d_ff // tf

    out = pl.pallas_call(
        functools.partial(_moe_dense_body, n_e=n_e, n_f=n_f),
        grid=(n_tm, n_e, n_f),
        in_specs=[
            pl.BlockSpec((tm, d), lambda i, e, f: (i, 0)),
            pl.BlockSpec((d, n_e), lambda i, e, f: (0, 0)),
            pl.BlockSpec((1, d, tf), lambda i, e, f: (e, 0, f)),
            pl.BlockSpec((1, d, tf), lambda i, e, f: (e, 0, f)),
            pl.BlockSpec((1, tf, d), lambda i, e, f: (e, f, 0)),
        ],
        out_specs=pl.BlockSpec((tm, d), lambda i, e, f: (i, 0)),
        out_shape=jax.ShapeDtypeStruct((b * s, d), jnp.float32),
        scratch_shapes=[
            pltpu.VMEM((tm, d), jnp.float32),
            pltpu.VMEM((tm, n_e), jnp.float32),
        ],
        compiler_params=pltpu.CompilerParams(
            dimension_semantics=("parallel", "arbitrary", "arbitrary"),
        ),
    )(xs, gate_kernel, w0, w1, wo)
    return out.reshape(b, s, d)
